# Initial kernel scaffold; baseline (speedup 1.0000x reference)
import jax
import jax.numpy as jnp
from jax.experimental import pallas as pl


def kernel(indices, counts, emb, w_ih, w_hh, b_ih, b_hh):
    raise NotImplementedError("write your pallas kernel here")


# speedup vs baseline: 3.1574x; 3.1574x over previous
import functools

import jax
import jax.numpy as jnp
from jax import lax
from jax.experimental import pallas as pl
from jax.experimental.pallas import tpu as pltpu
from jax.experimental.pallas import tpu_sc as plsc

_BN = 200
_CH = 80


def _sc_gather(emb, idx_flat):
    B = idx_flat.shape[0]
    D = emb.shape[1]
    info = plsc.get_sparse_core_info()
    nc, ns = info.num_cores, info.num_subcores
    nw = nc * ns
    b_per_w = B // nw
    n_chunks = b_per_w // _CH

    def body(table_hbm, idx_hbm, out_hbm, idx_v, rows_v, sem):
        wid = lax.axis_index("s") * nc + lax.axis_index("c")
        base = wid * b_per_w

        def step(i, carry):
            off = base + i * _CH
            pltpu.sync_copy(idx_hbm.at[pl.ds(off, _CH)], idx_v)
            pltpu.async_copy(table_hbm.at[idx_v], rows_v, sem).wait()
            pltpu.sync_copy(rows_v, out_hbm.at[pl.ds(off, _CH)])
            return carry

        lax.fori_loop(0, n_chunks, step, 0)

    gk = pl.kernel(
        body,
        out_type=jax.ShapeDtypeStruct((B, D), jnp.float32),
        mesh=plsc.VectorSubcoreMesh(core_axis_name="c", subcore_axis_name="s"),
        scratch_types=[
            pltpu.VMEM((_CH,), jnp.int32),
            pltpu.VMEM((_CH, D), jnp.float32),
            pltpu.SemaphoreType.DMA,
        ],
    )
    return gk(emb, idx_flat)


def _tc_body(L, D, emb_ref, c_ref, wih_ref, whh_ref, bih_ref, bhh_ref, out_ref):
    bn = c_ref.shape[0]
    x = emb_ref[...]
    dn = (((1,), (1,)), ((), ()))
    gi = lax.dot_general(x, wih_ref[...], dimension_numbers=dn,
                         preferred_element_type=jnp.float32) + bih_ref[...]
    i_r = gi[:, :D]
    i_z = gi[:, D:2 * D]
    i_n = gi[:, 2 * D:]
    bh = bhh_ref[...]

    c = c_ref[...]
    cn = c / jnp.sum(c, axis=1, keepdims=True)
    c3 = cn[:, :, None]

    r = jax.nn.sigmoid(i_r + bh[:, :D])
    z = jax.nn.sigmoid(i_z + bh[:, D:2 * D])
    n = jnp.tanh(i_n + r * bh[:, 2 * D:])
    hu = (1.0 - z) * n
    h1 = jnp.sum(hu.reshape(bn, L, D) * c3, axis=1)

    gh = lax.dot_general(h1, whh_ref[...], dimension_numbers=dn,
                         preferred_element_type=jnp.float32) + bh
    gh_b = jnp.broadcast_to(gh[:, None, :], (bn, L, 3 * D)).reshape(bn * L, 3 * D)
    h1_b = jnp.broadcast_to(h1[:, None, :], (bn, L, D)).reshape(bn * L, D)
    r2 = jax.nn.sigmoid(i_r + gh_b[:, :D])
    z2 = jax.nn.sigmoid(i_z + gh_b[:, D:2 * D])
    n2 = jnp.tanh(i_n + r2 * gh_b[:, 2 * D:])
    hu2 = (1.0 - z2) * n2 + z2 * h1_b
    out_ref[...] = jnp.sum(hu2.reshape(bn, L, D) * c3, axis=1)


def _tc_compute(emb_rows, counts, w_ih, w_hh, b_ih2, b_hh2, interpret=False):
    n_nodes, L = counts.shape
    D = emb_rows.shape[1]
    grid = (n_nodes // _BN,)
    return pl.pallas_call(
        functools.partial(_tc_body, L, D),
        grid=grid,
        in_specs=[
            pl.BlockSpec((_BN * L, D), lambda i: (i, 0)),
            pl.BlockSpec((_BN, L), lambda i: (i, 0)),
            pl.BlockSpec((3 * D, D), lambda i: (0, 0)),
            pl.BlockSpec((3 * D, D), lambda i: (0, 0)),
            pl.BlockSpec((1, 3 * D), lambda i: (0, 0)),
            pl.BlockSpec((1, 3 * D), lambda i: (0, 0)),
        ],
        out_specs=pl.BlockSpec((_BN, D), lambda i: (i, 0)),
        out_shape=jax.ShapeDtypeStruct((n_nodes, D), jnp.float32),
        interpret=interpret,
    )(emb_rows, counts, w_ih, w_hh, b_ih2, b_hh2)


def kernel(indices, counts, emb, w_ih, w_hh, b_ih, b_hh):
    idx_flat = indices.reshape(-1).astype(jnp.int32)
    emb_rows = _sc_gather(emb, idx_flat)
    return _tc_compute(emb_rows, counts, w_ih, w_hh,
                       b_ih.reshape(1, -1), b_hh.reshape(1, -1))


# speedup vs baseline: 4.2927x; 1.3595x over previous
import functools

import jax
import jax.numpy as jnp
from jax import lax
from jax.experimental import pallas as pl
from jax.experimental.pallas import tpu as pltpu
from jax.experimental.pallas import tpu_sc as plsc

_BN = 200
_CH = 80


_NBUF = 5


def _sc_gather(emb, idx_flat):
    B = idx_flat.shape[0]
    D = emb.shape[1]
    info = plsc.get_sparse_core_info()
    nc, ns = info.num_cores, info.num_subcores
    nw = nc * ns
    b_per_w = B // nw
    n_chunks = b_per_w // _CH

    def body(table_hbm, idx_hbm, out_hbm, idx_all, rows, *sems):
        sem_g = sems[:_NBUF]
        sem_w = sems[_NBUF:]
        wid = lax.axis_index("s") * nc + lax.axis_index("c")
        base = wid * b_per_w
        pltpu.sync_copy(idx_hbm.at[pl.ds(base, b_per_w)], idx_all)

        def gather(g, b):
            pltpu.async_copy(
                table_hbm.at[idx_all.at[pl.ds(g * _CH, _CH)]], rows.at[b],
                sem_g[b])

        def writeback(g, b):
            pltpu.async_copy(
                rows.at[b], out_hbm.at[pl.ds(base + g * _CH, _CH)], sem_w[b])

        def wait_gather(b):
            pltpu.make_async_copy(
                table_hbm.at[idx_all.at[pl.ds(0, _CH)]], rows.at[b],
                sem_g[b]).wait()

        def wait_write(b):
            pltpu.make_async_copy(
                rows.at[b], out_hbm.at[pl.ds(base, _CH)], sem_w[b]).wait()

        gather(0, 0)
        for g in range(1, _NBUF):
            b = g % _NBUF
            gather(g, b)
            bp = (b - 1) % _NBUF
            wait_gather(bp)
            writeback(g - 1, bp)

        def step(g2, carry):
            for b in range(_NBUF):
                g = g2 * _NBUF + b
                wait_write(b)
                gather(g, b)
                bp = (b - 1) % _NBUF
                wait_gather(bp)
                writeback(g - 1, bp)
            return carry

        lax.fori_loop(1, n_chunks // _NBUF, step, 0)

        last = n_chunks - 1
        bl = last % _NBUF
        wait_gather(bl)
        writeback(last, bl)
        for b in range(_NBUF):
            wait_write(b)

    gk = pl.kernel(
        body,
        out_type=jax.ShapeDtypeStruct((B, D), jnp.float32),
        mesh=plsc.VectorSubcoreMesh(core_axis_name="c", subcore_axis_name="s"),
        scratch_types=(
            [pltpu.VMEM((b_per_w,), jnp.int32),
             pltpu.VMEM((_NBUF, _CH, D), jnp.float32)]
            + [pltpu.SemaphoreType.DMA] * (2 * _NBUF)
        ),
    )
    return gk(emb, idx_flat)


def _tc_body(L, D, emb_ref, c_ref, wih_ref, whh_ref, bih_ref, bhh_ref, out_ref):
    bn = c_ref.shape[0]
    x = emb_ref[...]
    dn = (((1,), (1,)), ((), ()))
    gi = lax.dot_general(x, wih_ref[...], dimension_numbers=dn,
                         preferred_element_type=jnp.float32) + bih_ref[...]
    i_r = gi[:, :D]
    i_z = gi[:, D:2 * D]
    i_n = gi[:, 2 * D:]
    bh = bhh_ref[...]

    c = c_ref[...]
    cn = c / jnp.sum(c, axis=1, keepdims=True)
    c3 = cn[:, :, None]

    r = jax.nn.sigmoid(i_r + bh[:, :D])
    z = jax.nn.sigmoid(i_z + bh[:, D:2 * D])
    n = jnp.tanh(i_n + r * bh[:, 2 * D:])
    hu = (1.0 - z) * n
    h1 = jnp.sum(hu.reshape(bn, L, D) * c3, axis=1)

    gh = lax.dot_general(h1, whh_ref[...], dimension_numbers=dn,
                         preferred_element_type=jnp.float32) + bh
    gh_b = jnp.broadcast_to(gh[:, None, :], (bn, L, 3 * D)).reshape(bn * L, 3 * D)
    h1_b = jnp.broadcast_to(h1[:, None, :], (bn, L, D)).reshape(bn * L, D)
    r2 = jax.nn.sigmoid(i_r + gh_b[:, :D])
    z2 = jax.nn.sigmoid(i_z + gh_b[:, D:2 * D])
    n2 = jnp.tanh(i_n + r2 * gh_b[:, 2 * D:])
    hu2 = (1.0 - z2) * n2 + z2 * h1_b
    out_ref[...] = jnp.sum(hu2.reshape(bn, L, D) * c3, axis=1)


def _tc_compute(emb_rows, counts, w_ih, w_hh, b_ih2, b_hh2, interpret=False):
    n_nodes, L = counts.shape
    D = emb_rows.shape[1]
    grid = (n_nodes // _BN,)
    return pl.pallas_call(
        functools.partial(_tc_body, L, D),
        grid=grid,
        in_specs=[
            pl.BlockSpec((_BN * L, D), lambda i: (i, 0)),
            pl.BlockSpec((_BN, L), lambda i: (i, 0)),
            pl.BlockSpec((3 * D, D), lambda i: (0, 0)),
            pl.BlockSpec((3 * D, D), lambda i: (0, 0)),
            pl.BlockSpec((1, 3 * D), lambda i: (0, 0)),
            pl.BlockSpec((1, 3 * D), lambda i: (0, 0)),
        ],
        out_specs=pl.BlockSpec((_BN, D), lambda i: (i, 0)),
        out_shape=jax.ShapeDtypeStruct((n_nodes, D), jnp.float32),
        interpret=interpret,
    )(emb_rows, counts, w_ih, w_hh, b_ih2, b_hh2)


def kernel(indices, counts, emb, w_ih, w_hh, b_ih, b_hh):
    idx_flat = indices.reshape(-1).astype(jnp.int32)
    emb_rows = _sc_gather(emb, idx_flat)
    return _tc_compute(emb_rows, counts, w_ih, w_hh,
                       b_ih.reshape(1, -1), b_hh.reshape(1, -1))


# speedup vs baseline: 4.7844x; 1.1145x over previous
import functools

import jax
import jax.numpy as jnp
from jax import lax
from jax.experimental import pallas as pl
from jax.experimental.pallas import tpu as pltpu
from jax.experimental.pallas import tpu_sc as plsc

_BN = 200
_CH = 80


_NBUF = 5


def _sc_gather(emb, idx_flat):
    B = idx_flat.shape[0]
    D = emb.shape[1]
    info = plsc.get_sparse_core_info()
    nc, ns = info.num_cores, info.num_subcores
    nw = nc * ns
    b_per_w = B // nw
    n_chunks = b_per_w // _CH

    def body(table_hbm, idx_hbm, out_hbm, idx_all, rows, *sems):
        sem_g = sems[:_NBUF]
        sem_w = sems[_NBUF:]
        wid = lax.axis_index("s") * nc + lax.axis_index("c")
        base = wid * b_per_w
        pltpu.sync_copy(idx_hbm.at[pl.ds(base, b_per_w)], idx_all)

        def gather(g, b):
            pltpu.async_copy(
                table_hbm.at[idx_all.at[pl.ds(g * _CH, _CH)]], rows.at[b],
                sem_g[b])

        def writeback(g, b):
            pltpu.async_copy(
                rows.at[b], out_hbm.at[pl.ds(base + g * _CH, _CH)], sem_w[b])

        def wait_gather(b):
            pltpu.make_async_copy(
                table_hbm.at[idx_all.at[pl.ds(0, _CH)]], rows.at[b],
                sem_g[b]).wait()

        def wait_write(b):
            pltpu.make_async_copy(
                rows.at[b], out_hbm.at[pl.ds(base, _CH)], sem_w[b]).wait()

        gather(0, 0)
        for g in range(1, _NBUF):
            b = g % _NBUF
            gather(g, b)
            bp = (b - 1) % _NBUF
            wait_gather(bp)
            writeback(g - 1, bp)

        def step(g2, carry):
            for b in range(_NBUF):
                g = g2 * _NBUF + b
                wait_write(b)
                gather(g, b)
                bp = (b - 1) % _NBUF
                wait_gather(bp)
                writeback(g - 1, bp)
            return carry

        lax.fori_loop(1, n_chunks // _NBUF, step, 0)

        last = n_chunks - 1
        bl = last % _NBUF
        wait_gather(bl)
        writeback(last, bl)
        for b in range(_NBUF):
            wait_write(b)

    gk = pl.kernel(
        body,
        out_type=jax.ShapeDtypeStruct((B, D), jnp.float32),
        mesh=plsc.VectorSubcoreMesh(core_axis_name="c", subcore_axis_name="s"),
        scratch_types=(
            [pltpu.VMEM((b_per_w,), jnp.int32),
             pltpu.VMEM((_NBUF, _CH, D), jnp.float32)]
            + [pltpu.SemaphoreType.DMA] * (2 * _NBUF)
        ),
    )
    return gk(emb, idx_flat)


def _tc_body(L, D, emb_ref, c_ref, wih_ref, whh_ref, c1_ref, c2_ref,
             dn1_ref, bihn_ref, out_ref):
    bn = c_ref.shape[0]
    x = emb_ref[...]
    dn = (((1,), (1,)), ((), ()))
    g = lax.dot_general(x, wih_ref[...], dimension_numbers=dn,
                        preferred_element_type=jnp.float32)
    g_r = g[:, :D]
    g_z = g[:, D:2 * D]
    g_n = g[:, 2 * D:]
    c1 = c1_ref[...]

    c = c_ref[...]
    cn_half = c * (0.5 / jnp.sum(c, axis=1, keepdims=True))
    c3 = cn_half[:, :, None]

    tr1 = jnp.tanh(g_r + c1[:, :D])
    tz1 = jnp.tanh(g_z + c1[:, D:2 * D])
    tn1 = jnp.tanh((g_n + c1[:, 2 * D:]) + dn1_ref[...] * tr1)
    v1 = (1.0 - tz1) * tn1
    h1 = jnp.sum(v1.reshape(bn, L, D) * c3, axis=1)

    gh2 = lax.dot_general(h1, whh_ref[...], dimension_numbers=dn,
                          preferred_element_type=jnp.float32) + c2_ref[...]
    p = 0.5 * gh2[:, 2 * D:]
    ghrz_b = jnp.broadcast_to(gh2[:, None, :2 * D],
                              (bn, L, 2 * D)).reshape(bn * L, 2 * D)
    p_b = jnp.broadcast_to(p[:, None, :], (bn, L, D)).reshape(bn * L, D)
    h1_b = jnp.broadcast_to(h1[:, None, :], (bn, L, D)).reshape(bn * L, D)
    tr2 = jnp.tanh(g_r + ghrz_b[:, :D])
    tz2 = jnp.tanh(g_z + ghrz_b[:, D:])
    tn2 = jnp.tanh((g_n + bihn_ref[...]) + p_b * (1.0 + tr2))
    v2 = (1.0 - tz2) * tn2 + (1.0 + tz2) * h1_b
    out_ref[...] = jnp.sum(v2.reshape(bn, L, D) * c3, axis=1)


def _tc_compute(emb_rows, counts, wih_s, whh_s, c1, c2, dn1, bihn,
                interpret=False):
    n_nodes, L = counts.shape
    D = emb_rows.shape[1]
    grid = (n_nodes // _BN,)
    return pl.pallas_call(
        functools.partial(_tc_body, L, D),
        grid=grid,
        in_specs=[
            pl.BlockSpec((_BN * L, D), lambda i: (i, 0)),
            pl.BlockSpec((_BN, L), lambda i: (i, 0)),
            pl.BlockSpec((3 * D, D), lambda i: (0, 0)),
            pl.BlockSpec((3 * D, D), lambda i: (0, 0)),
            pl.BlockSpec((1, 3 * D), lambda i: (0, 0)),
            pl.BlockSpec((1, 3 * D), lambda i: (0, 0)),
            pl.BlockSpec((1, D), lambda i: (0, 0)),
            pl.BlockSpec((1, D), lambda i: (0, 0)),
        ],
        out_specs=pl.BlockSpec((_BN, D), lambda i: (i, 0)),
        out_shape=jax.ShapeDtypeStruct((n_nodes, D), jnp.float32),
        interpret=interpret,
    )(emb_rows, counts, wih_s, whh_s, c1, c2, dn1, bihn)


def _fold_params(w_ih, w_hh, b_ih, b_hh):
    D = w_ih.shape[1]
    half = jnp.concatenate([jnp.full((2 * D, 1), 0.5, jnp.float32),
                            jnp.ones((D, 1), jnp.float32)], axis=0)
    wih_s = w_ih * half
    whh_s = w_hh * half
    bir, biz, bin_ = b_ih[:D], b_ih[D:2 * D], b_ih[2 * D:]
    bhr, bhz, bhn = b_hh[:D], b_hh[D:2 * D], b_hh[2 * D:]
    rz = jnp.concatenate([0.5 * (bir + bhr), 0.5 * (biz + bhz)])
    c1 = jnp.concatenate([rz, bin_ + 0.5 * bhn]).reshape(1, 3 * D)
    c2 = jnp.concatenate([rz, bhn]).reshape(1, 3 * D)
    dn1 = (0.5 * bhn).reshape(1, D)
    bihn = bin_.reshape(1, D)
    return wih_s, whh_s, c1, c2, dn1, bihn


def kernel(indices, counts, emb, w_ih, w_hh, b_ih, b_hh):
    idx_flat = indices.reshape(-1).astype(jnp.int32)
    emb_rows = _sc_gather(emb, idx_flat)
    return _tc_compute(emb_rows, counts, *_fold_params(w_ih, w_hh, b_ih, b_hh))


# speedup vs baseline: 5.7790x; 1.2079x over previous
import functools

import jax
import jax.numpy as jnp
from jax import lax
from jax.experimental import pallas as pl
from jax.experimental.pallas import tpu as pltpu
from jax.experimental.pallas import tpu_sc as plsc

_BN = 200
_CH = 80


_NBUF = 5


def _sc_gather(emb, idx_flat):
    B = idx_flat.shape[0]
    D = emb.shape[1]
    info = plsc.get_sparse_core_info()
    nc, ns = info.num_cores, info.num_subcores
    nw = nc * ns
    b_per_w = B // nw
    n_chunks = b_per_w // _CH

    def body(table_hbm, idx_hbm, out_hbm, idx_all, rows, *sems):
        sem_g = sems[:_NBUF]
        sem_w = sems[_NBUF:]
        wid = lax.axis_index("s") * nc + lax.axis_index("c")
        base = wid * b_per_w
        pltpu.sync_copy(idx_hbm.at[pl.ds(base, b_per_w)], idx_all)

        def gather(g, b):
            pltpu.async_copy(
                table_hbm.at[idx_all.at[pl.ds(g * _CH, _CH)]], rows.at[b],
                sem_g[b])

        def writeback(g, b):
            pltpu.async_copy(
                rows.at[b], out_hbm.at[pl.ds(base + g * _CH, _CH)], sem_w[b])

        def wait_gather(b):
            pltpu.make_async_copy(
                table_hbm.at[idx_all.at[pl.ds(0, _CH)]], rows.at[b],
                sem_g[b]).wait()

        def wait_write(b):
            pltpu.make_async_copy(
                rows.at[b], out_hbm.at[pl.ds(base, _CH)], sem_w[b]).wait()

        gather(0, 0)
        for g in range(1, _NBUF):
            b = g % _NBUF
            gather(g, b)
            bp = (b - 1) % _NBUF
            wait_gather(bp)
            writeback(g - 1, bp)

        def step(g2, carry):
            for b in range(_NBUF):
                g = g2 * _NBUF + b
                wait_write(b)
                gather(g, b)
                bp = (b - 1) % _NBUF
                wait_gather(bp)
                writeback(g - 1, bp)
            return carry

        lax.fori_loop(1, n_chunks // _NBUF, step, 0)

        last = n_chunks - 1
        bl = last % _NBUF
        wait_gather(bl)
        writeback(last, bl)
        for b in range(_NBUF):
            wait_write(b)

    gk = pl.kernel(
        body,
        out_type=jax.ShapeDtypeStruct((B, D), jnp.float32),
        mesh=plsc.VectorSubcoreMesh(core_axis_name="c", subcore_axis_name="s"),
        scratch_types=(
            [pltpu.VMEM((b_per_w,), jnp.int32),
             pltpu.VMEM((_NBUF, _CH, D), jnp.float32)]
            + [pltpu.SemaphoreType.DMA] * (2 * _NBUF)
        ),
    )
    return gk(emb, idx_flat)


def _tc_body(L, D, emb_ref, c_ref, wih_ref, whh_ref, c1_ref, c2_ref,
             dn1_ref, bihn_ref, out_ref):
    bn = c_ref.shape[0]
    x = emb_ref[...]
    dn = (((1,), (1,)), ((), ()))
    g = lax.dot_general(x, wih_ref[...], dimension_numbers=dn,
                        preferred_element_type=jnp.float32)
    g_r = g[:, :D]
    g_z = g[:, D:2 * D]
    g_n = g[:, 2 * D:]
    c1 = c1_ref[...]

    c = c_ref[...]
    cn_half = c * (0.5 / jnp.sum(c, axis=1, keepdims=True))
    c3 = cn_half[:, :, None]

    tr1 = jnp.tanh(g_r + c1[:, :D])
    tz1 = jnp.tanh(g_z + c1[:, D:2 * D])
    tn1 = jnp.tanh((g_n + c1[:, 2 * D:]) + dn1_ref[...] * tr1)
    v1 = (1.0 - tz1) * tn1
    h1 = jnp.sum(v1.reshape(bn, L, D) * c3, axis=1)

    gh2 = lax.dot_general(h1, whh_ref[...], dimension_numbers=dn,
                          preferred_element_type=jnp.float32) + c2_ref[...]
    p = 0.5 * gh2[:, 2 * D:]
    ghrz_b = jnp.broadcast_to(gh2[:, None, :2 * D],
                              (bn, L, 2 * D)).reshape(bn * L, 2 * D)
    p_b = jnp.broadcast_to(p[:, None, :], (bn, L, D)).reshape(bn * L, D)
    h1_b = jnp.broadcast_to(h1[:, None, :], (bn, L, D)).reshape(bn * L, D)
    tr2 = jnp.tanh(g_r + ghrz_b[:, :D])
    tz2 = jnp.tanh(g_z + ghrz_b[:, D:])
    tn2 = jnp.tanh((g_n + bihn_ref[...]) + p_b * (1.0 + tr2))
    v2 = (1.0 - tz2) * tn2 + (1.0 + tz2) * h1_b
    out_ref[...] = jnp.sum(v2.reshape(bn, L, D) * c3, axis=1)


def _tc_compute(emb_rows, counts, wih_s, whh_s, c1, c2, dn1, bihn,
                interpret=False):
    n_nodes, L = counts.shape
    D = emb_rows.shape[1]
    grid = (n_nodes // _BN,)
    return pl.pallas_call(
        functools.partial(_tc_body, L, D),
        grid=grid,
        in_specs=[
            pl.BlockSpec((_BN * L, D), lambda i: (i, 0)),
            pl.BlockSpec((_BN, L), lambda i: (i, 0)),
            pl.BlockSpec((3 * D, D), lambda i: (0, 0)),
            pl.BlockSpec((3 * D, D), lambda i: (0, 0)),
            pl.BlockSpec((1, 3 * D), lambda i: (0, 0)),
            pl.BlockSpec((1, 3 * D), lambda i: (0, 0)),
            pl.BlockSpec((1, D), lambda i: (0, 0)),
            pl.BlockSpec((1, D), lambda i: (0, 0)),
        ],
        out_specs=pl.BlockSpec((_BN, D), lambda i: (i, 0)),
        out_shape=jax.ShapeDtypeStruct((n_nodes, D), jnp.float32),
        interpret=interpret,
    )(emb_rows, counts, wih_s, whh_s, c1, c2, dn1, bihn)


def _fold_params(w_ih, w_hh, b_ih, b_hh):
    D = w_ih.shape[1]
    half = jnp.concatenate([jnp.full((2 * D, 1), 0.5, jnp.float32),
                            jnp.ones((D, 1), jnp.float32)], axis=0)
    wih_s = w_ih * half
    whh_s = w_hh * half
    bir, biz, bin_ = b_ih[:D], b_ih[D:2 * D], b_ih[2 * D:]
    bhr, bhz, bhn = b_hh[:D], b_hh[D:2 * D], b_hh[2 * D:]
    rz = jnp.concatenate([0.5 * (bir + bhr), 0.5 * (biz + bhz)])
    c1 = jnp.concatenate([rz, bin_ + 0.5 * bhn]).reshape(1, 3 * D)
    c2 = jnp.concatenate([rz, bhn]).reshape(1, 3 * D)
    dn1 = (0.5 * bhn).reshape(1, D)
    bihn = bin_.reshape(1, D)
    return wih_s, whh_s, c1, c2, dn1, bihn


_NSLICE = 5


def kernel(indices, counts, emb, w_ih, w_hh, b_ih, b_hh):
    n_nodes, L = indices.shape
    idx_flat = indices.reshape(-1).astype(jnp.int32)
    folded = _fold_params(w_ih, w_hh, b_ih, b_hh)
    ns = n_nodes // _NSLICE
    outs = []
    for s in range(_NSLICE):
        emb_rows = _sc_gather(emb, idx_flat[s * ns * L:(s + 1) * ns * L])
        outs.append(_tc_compute(emb_rows, counts[s * ns:(s + 1) * ns], *folded))
    return jnp.concatenate(outs, axis=0)


# speedup vs baseline: 5.7844x; 1.0009x over previous
import functools

import jax
import jax.numpy as jnp
from jax import lax
from jax.experimental import pallas as pl
from jax.experimental.pallas import tpu as pltpu
from jax.experimental.pallas import tpu_sc as plsc

_BN = 200
_CH = 80


_NBUF = 5


def _sc_gather(emb, idx_flat):
    B = idx_flat.shape[0]
    D = emb.shape[1]
    info = plsc.get_sparse_core_info()
    nc, ns = info.num_cores, info.num_subcores
    nw = nc * ns
    b_per_w = B // nw
    n_chunks = b_per_w // _CH

    def body(table_hbm, idx_hbm, out_hbm, idx_all, rows, *sems):
        sem_g = sems[:_NBUF]
        sem_w = sems[_NBUF:]
        wid = lax.axis_index("s") * nc + lax.axis_index("c")
        base = wid * b_per_w
        pltpu.sync_copy(idx_hbm.at[pl.ds(base, b_per_w)], idx_all)

        def gather(g, b):
            pltpu.async_copy(
                table_hbm.at[idx_all.at[pl.ds(g * _CH, _CH)]], rows.at[b],
                sem_g[b])

        def writeback(g, b):
            pltpu.async_copy(
                rows.at[b], out_hbm.at[pl.ds(base + g * _CH, _CH)], sem_w[b])

        def wait_gather(b):
            pltpu.make_async_copy(
                table_hbm.at[idx_all.at[pl.ds(0, _CH)]], rows.at[b],
                sem_g[b]).wait()

        def wait_write(b):
            pltpu.make_async_copy(
                rows.at[b], out_hbm.at[pl.ds(base, _CH)], sem_w[b]).wait()

        gather(0, 0)
        for g in range(1, _NBUF):
            b = g % _NBUF
            gather(g, b)
            bp = (b - 1) % _NBUF
            wait_gather(bp)
            writeback(g - 1, bp)

        def step(g2, carry):
            for b in range(_NBUF):
                g = g2 * _NBUF + b
                wait_write(b)
                gather(g, b)
                bp = (b - 1) % _NBUF
                wait_gather(bp)
                writeback(g - 1, bp)
            return carry

        lax.fori_loop(1, n_chunks // _NBUF, step, 0)

        last = n_chunks - 1
        bl = last % _NBUF
        wait_gather(bl)
        writeback(last, bl)
        for b in range(_NBUF):
            wait_write(b)

    gk = pl.kernel(
        body,
        out_type=jax.ShapeDtypeStruct((B, D), jnp.float32),
        mesh=plsc.VectorSubcoreMesh(core_axis_name="c", subcore_axis_name="s"),
        scratch_types=(
            [pltpu.VMEM((b_per_w,), jnp.int32),
             pltpu.VMEM((_NBUF, _CH, D), jnp.float32)]
            + [pltpu.SemaphoreType.DMA] * (2 * _NBUF)
        ),
    )
    return gk(emb, idx_flat)


def _tc_body(L, D, emb_ref, c_ref, wih_ref, whh_ref, c1_ref, c2_ref,
             dn1_ref, bihn_ref, out_ref):
    bn = c_ref.shape[0]
    x = emb_ref[...].reshape(L * bn, D)
    dn = (((1,), (1,)), ((), ()))
    g = lax.dot_general(x, wih_ref[...], dimension_numbers=dn,
                        preferred_element_type=jnp.float32)
    c1 = c1_ref[...]

    c = c_ref[...]
    cn_half = c * (0.5 / jnp.sum(c, axis=1, keepdims=True))

    tr1 = jnp.tanh(g[:, :D] + c1[:, :D])
    tz1 = jnp.tanh(g[:, D:2 * D] + c1[:, D:2 * D])
    tn1 = jnp.tanh((g[:, 2 * D:] + c1[:, 2 * D:]) + dn1_ref[...] * tr1)
    v1 = (1.0 - tz1) * tn1
    h1 = jnp.zeros((bn, D), jnp.float32)
    for l in range(L):
        h1 = h1 + v1[l * bn:(l + 1) * bn] * cn_half[:, l:l + 1]

    gh2 = lax.dot_general(h1, whh_ref[...], dimension_numbers=dn,
                          preferred_element_type=jnp.float32) + c2_ref[...]
    ghr = gh2[:, :D]
    ghz = gh2[:, D:2 * D]
    p = 0.5 * gh2[:, 2 * D:]
    bihn = bihn_ref[...]
    out = jnp.zeros((bn, D), jnp.float32)
    for l in range(L):
        gl = g[l * bn:(l + 1) * bn]
        trl = jnp.tanh(gl[:, :D] + ghr)
        tzl = jnp.tanh(gl[:, D:2 * D] + ghz)
        tnl = jnp.tanh((gl[:, 2 * D:] + bihn) + p * (1.0 + trl))
        v2l = (1.0 - tzl) * tnl + (1.0 + tzl) * h1
        out = out + v2l * cn_half[:, l:l + 1]
    out_ref[...] = out


def _tc_compute(emb_rows_lm, counts, wih_s, whh_s, c1, c2, dn1, bihn,
                interpret=False):
    n_nodes, L = counts.shape
    D = emb_rows_lm.shape[1]
    emb3 = emb_rows_lm.reshape(L, n_nodes, D)
    grid = (n_nodes // _BN,)
    return pl.pallas_call(
        functools.partial(_tc_body, L, D),
        grid=grid,
        in_specs=[
            pl.BlockSpec((L, _BN, D), lambda i: (0, i, 0)),
            pl.BlockSpec((_BN, L), lambda i: (i, 0)),
            pl.BlockSpec((3 * D, D), lambda i: (0, 0)),
            pl.BlockSpec((3 * D, D), lambda i: (0, 0)),
            pl.BlockSpec((1, 3 * D), lambda i: (0, 0)),
            pl.BlockSpec((1, 3 * D), lambda i: (0, 0)),
            pl.BlockSpec((1, D), lambda i: (0, 0)),
            pl.BlockSpec((1, D), lambda i: (0, 0)),
        ],
        out_specs=pl.BlockSpec((_BN, D), lambda i: (i, 0)),
        out_shape=jax.ShapeDtypeStruct((n_nodes, D), jnp.float32),
        interpret=interpret,
    )(emb3, counts, wih_s, whh_s, c1, c2, dn1, bihn)


def _fold_params(w_ih, w_hh, b_ih, b_hh):
    D = w_ih.shape[1]
    half = jnp.concatenate([jnp.full((2 * D, 1), 0.5, jnp.float32),
                            jnp.ones((D, 1), jnp.float32)], axis=0)
    wih_s = w_ih * half
    whh_s = w_hh * half
    bir, biz, bin_ = b_ih[:D], b_ih[D:2 * D], b_ih[2 * D:]
    bhr, bhz, bhn = b_hh[:D], b_hh[D:2 * D], b_hh[2 * D:]
    rz = jnp.concatenate([0.5 * (bir + bhr), 0.5 * (biz + bhz)])
    c1 = jnp.concatenate([rz, bin_ + 0.5 * bhn]).reshape(1, 3 * D)
    c2 = jnp.concatenate([rz, bhn]).reshape(1, 3 * D)
    dn1 = (0.5 * bhn).reshape(1, D)
    bihn = bin_.reshape(1, D)
    return wih_s, whh_s, c1, c2, dn1, bihn


_NSLICE = 5


def kernel(indices, counts, emb, w_ih, w_hh, b_ih, b_hh):
    n_nodes, L = indices.shape
    idx_lm = indices.T.astype(jnp.int32)
    folded = _fold_params(w_ih, w_hh, b_ih, b_hh)
    ns = n_nodes // _NSLICE
    outs = []
    for s in range(_NSLICE):
        idx_s = idx_lm[:, s * ns:(s + 1) * ns].reshape(-1)
        emb_rows = _sc_gather(emb, idx_s)
        outs.append(_tc_compute(emb_rows, counts[s * ns:(s + 1) * ns], *folded))
    return jnp.concatenate(outs, axis=0)


# speedup vs baseline: 6.0068x; 1.0384x over previous
import functools

import jax
import jax.numpy as jnp
from jax import lax
from jax.experimental import pallas as pl
from jax.experimental.pallas import tpu as pltpu
from jax.experimental.pallas import tpu_sc as plsc

_BN = 200
_NT = 40
_CH = 80


_NBUF = 5


def _sc_gather(emb, idx_flat):
    B = idx_flat.shape[0]
    D = emb.shape[1]
    info = plsc.get_sparse_core_info()
    nc, ns = info.num_cores, info.num_subcores
    nw = nc * ns
    b_per_w = B // nw
    n_chunks = b_per_w // _CH

    def body(table_hbm, idx_hbm, out_hbm, idx_all, rows, *sems):
        sem_g = sems[:_NBUF]
        sem_w = sems[_NBUF:]
        wid = lax.axis_index("s") * nc + lax.axis_index("c")
        base = wid * b_per_w
        pltpu.sync_copy(idx_hbm.at[pl.ds(base, b_per_w)], idx_all)

        def gather(g, b):
            pltpu.async_copy(
                table_hbm.at[idx_all.at[pl.ds(g * _CH, _CH)]], rows.at[b],
                sem_g[b])

        def writeback(g, b):
            pltpu.async_copy(
                rows.at[b], out_hbm.at[pl.ds(base + g * _CH, _CH)], sem_w[b])

        def wait_gather(b):
            pltpu.make_async_copy(
                table_hbm.at[idx_all.at[pl.ds(0, _CH)]], rows.at[b],
                sem_g[b]).wait()

        def wait_write(b):
            pltpu.make_async_copy(
                rows.at[b], out_hbm.at[pl.ds(base, _CH)], sem_w[b]).wait()

        gather(0, 0)
        for g in range(1, _NBUF):
            b = g % _NBUF
            gather(g, b)
            bp = (b - 1) % _NBUF
            wait_gather(bp)
            writeback(g - 1, bp)

        def step(g2, carry):
            for b in range(_NBUF):
                g = g2 * _NBUF + b
                wait_write(b)
                gather(g, b)
                bp = (b - 1) % _NBUF
                wait_gather(bp)
                writeback(g - 1, bp)
            return carry

        lax.fori_loop(1, n_chunks // _NBUF, step, 0)

        last = n_chunks - 1
        bl = last % _NBUF
        wait_gather(bl)
        writeback(last, bl)
        for b in range(_NBUF):
            wait_write(b)

    gk = pl.kernel(
        body,
        out_type=jax.ShapeDtypeStruct((B, D), jnp.float32),
        mesh=plsc.VectorSubcoreMesh(core_axis_name="c", subcore_axis_name="s"),
        scratch_types=(
            [pltpu.VMEM((b_per_w,), jnp.int32),
             pltpu.VMEM((_NBUF, _CH, D), jnp.float32)]
            + [pltpu.SemaphoreType.DMA] * (2 * _NBUF)
        ),
    )
    return gk(emb, idx_flat)


def _tc_body(L, D, emb_ref, c_ref, wih_ref, whh_ref, c1_ref, c2_ref,
             dn1_ref, bihn_ref, out_ref):
    bn = c_ref.shape[0]
    x = emb_ref[...].reshape(L * bn, D)
    dn = (((1,), (1,)), ((), ()))
    g = lax.dot_general(x, wih_ref[...], dimension_numbers=dn,
                        preferred_element_type=jnp.float32)
    c1 = c1_ref[...]

    c = c_ref[...]
    cn_half = c * (0.5 / jnp.sum(c, axis=1, keepdims=True))

    bf = jnp.bfloat16
    gb = g.astype(bf)
    c1b = c1.astype(bf)
    dn1b = dn1_ref[...].astype(bf)

    tr1 = jnp.tanh(gb[:, :D] + c1b[:, :D])
    tz1 = jnp.tanh(gb[:, D:2 * D] + c1b[:, D:2 * D])
    tn1 = jnp.tanh((gb[:, 2 * D:] + c1b[:, 2 * D:]) + dn1b * tr1)
    v1 = (tn1 - tz1 * tn1).astype(jnp.float32)
    h1acc = [jnp.zeros((bn, D), jnp.float32) for _ in range(4)]
    for l in range(L):
        h1acc[l % 4] = (h1acc[l % 4]
                        + v1[l * bn:(l + 1) * bn] * cn_half[:, l:l + 1])
    h1 = (h1acc[0] + h1acc[1]) + (h1acc[2] + h1acc[3])

    gh2 = lax.dot_general(h1, whh_ref[...], dimension_numbers=dn,
                          preferred_element_type=jnp.float32) + c2_ref[...]
    ghr = gh2[:, :D].astype(bf)
    ghz = gh2[:, D:2 * D].astype(bf)
    p = (0.5 * gh2[:, 2 * D:]).astype(bf)
    q = bihn_ref[...].astype(bf) + p
    h1b = h1.astype(bf)
    oacc = [jnp.zeros((bn, D), jnp.float32) for _ in range(4)]
    for l in range(L):
        gl = gb[l * bn:(l + 1) * bn]
        trl = jnp.tanh(gl[:, :D] + ghr)
        tzl = jnp.tanh(gl[:, D:2 * D] + ghz)
        tnl = jnp.tanh((gl[:, 2 * D:] + q) + p * trl)
        v2l = ((tnl + h1b) + tzl * (h1b - tnl)).astype(jnp.float32)
        oacc[l % 4] = oacc[l % 4] + v2l * cn_half[:, l:l + 1]
    out_ref[...] = (oacc[0] + oacc[1]) + (oacc[2] + oacc[3])


def _tc_compute(emb_rows_lm, counts, wih_s, whh_s, c1, c2, dn1, bihn,
                interpret=False):
    n_nodes, L = counts.shape
    D = emb_rows_lm.shape[1]
    emb3 = emb_rows_lm.reshape(L, n_nodes, D)
    grid = (n_nodes // _BN,)
    return pl.pallas_call(
        functools.partial(_tc_body, L, D),
        grid=grid,
        in_specs=[
            pl.BlockSpec((L, _BN, D), lambda i: (0, i, 0)),
            pl.BlockSpec((_BN, L), lambda i: (i, 0)),
            pl.BlockSpec((3 * D, D), lambda i: (0, 0)),
            pl.BlockSpec((3 * D, D), lambda i: (0, 0)),
            pl.BlockSpec((1, 3 * D), lambda i: (0, 0)),
            pl.BlockSpec((1, 3 * D), lambda i: (0, 0)),
            pl.BlockSpec((1, D), lambda i: (0, 0)),
            pl.BlockSpec((1, D), lambda i: (0, 0)),
        ],
        out_specs=pl.BlockSpec((_BN, D), lambda i: (i, 0)),
        out_shape=jax.ShapeDtypeStruct((n_nodes, D), jnp.float32),
        interpret=interpret,
    )(emb3, counts, wih_s, whh_s, c1, c2, dn1, bihn)


def _fold_params(w_ih, w_hh, b_ih, b_hh):
    D = w_ih.shape[1]
    half = jnp.concatenate([jnp.full((2 * D, 1), 0.5, jnp.float32),
                            jnp.ones((D, 1), jnp.float32)], axis=0)
    wih_s = w_ih * half
    whh_s = w_hh * half
    bir, biz, bin_ = b_ih[:D], b_ih[D:2 * D], b_ih[2 * D:]
    bhr, bhz, bhn = b_hh[:D], b_hh[D:2 * D], b_hh[2 * D:]
    rz = jnp.concatenate([0.5 * (bir + bhr), 0.5 * (biz + bhz)])
    c1 = jnp.concatenate([rz, bin_ + 0.5 * bhn]).reshape(1, 3 * D)
    c2 = jnp.concatenate([rz, bhn]).reshape(1, 3 * D)
    dn1 = (0.5 * bhn).reshape(1, D)
    bihn = bin_.reshape(1, D)
    return wih_s, whh_s, c1, c2, dn1, bihn


_NSLICE = 5


def kernel(indices, counts, emb, w_ih, w_hh, b_ih, b_hh):
    n_nodes, L = indices.shape
    idx_lm = indices.T.astype(jnp.int32)
    folded = _fold_params(w_ih, w_hh, b_ih, b_hh)
    ns = n_nodes // _NSLICE
    outs = []
    for s in range(_NSLICE):
        idx_s = idx_lm[:, s * ns:(s + 1) * ns].reshape(-1)
        emb_rows = _sc_gather(emb, idx_s)
        outs.append(_tc_compute(emb_rows, counts[s * ns:(s + 1) * ns], *folded))
    return jnp.concatenate(outs, axis=0)
